# baseline (speedup 1.0000x reference)
import functools

import jax
import jax.numpy as jnp
from jax import lax
from jax.experimental import pallas as pl
from jax.experimental.pallas import tpu as pltpu
from jax.experimental.pallas import tpu_sc as plsc

NC = 2
NS = 16
NW = NC * NS
LANES = 16


def _make_transpose_kernel(V, D):
    TCOL = 128
    n_full = V // TCOL
    rem = V - n_full * TCOL
    per = n_full // NW
    extra = n_full - per * NW
    n_j = D // LANES

    mesh = plsc.VectorSubcoreMesh(
        core_axis_name="c", subcore_axis_name="s", num_cores=NC,
        num_subcores=NS)

    @functools.partial(
        pl.kernel,
        mesh=mesh,
        compiler_params=pltpu.CompilerParams(
            use_tc_tiling_on_sc=True, needs_layout_passes=False),
        out_type=jax.ShapeDtypeStruct((V * D,), jnp.float32),
        scratch_types=[
            pltpu.VMEM((D, TCOL), jnp.float32),
            pltpu.VMEM((D, TCOL), jnp.float32),
            pltpu.VMEM((TCOL * D,), jnp.float32),
            pltpu.VMEM((TCOL * D,), jnp.float32),
            pltpu.SemaphoreType.DMA,
            pltpu.SemaphoreType.DMA,
            pltpu.SemaphoreType.DMA,
        ],
    )
    def transpose_k(tT_hbm, tail_hbm, out_hbm, stage0, stage1, outb0, outb1,
                    semi0, semi1, semo):
        wid = lax.axis_index("s") * NC + lax.axis_index("c")
        cnt = per + jnp.where(wid < extra, 1, 0)
        start = per * wid + jnp.minimum(wid, extra)

        iota = lax.iota(jnp.int32, 16)
        idx_base = [(iota + LANES * g) * D for g in range(TCOL // LANES)]
        zvec = jnp.zeros((LANES,), jnp.int32) + wid * 0

        def fire(k, stage, sem):
            col0 = (start + k) * TCOL
            pltpu.async_copy(
                tT_hbm.at[:, pl.ds(col0, TCOL)], stage, sem)

        def drain(k, stage, sem):
            col0 = (start + k) * TCOL
            pltpu.make_async_copy(
                tT_hbm.at[:, pl.ds(col0, TCOL)], stage, sem).wait()

        def transpose_block(stage, outb):
            for d in range(D):
                dvec = zvec + d
                for g in range(TCOL // LANES):
                    v = stage[d, pl.ds(LANES * g, LANES)]
                    plsc.store_scatter(outb, [idx_base[g] + dvec], v)

        fire(0, stage0, semi0)

        def body2(i, _):
            k = i * 2

            @pl.when(k < cnt)
            def _():
                drain(k, stage0, semi0)

                @pl.when(k + 1 < cnt)
                def _():
                    fire(k + 1, stage1, semi1)

                transpose_block(stage0, outb0)
                col0 = (start + k) * TCOL
                pltpu.sync_copy(outb0, out_hbm.at[pl.ds(col0 * D, TCOL * D)])

            @pl.when(k + 1 < cnt)
            def _():
                drain(k + 1, stage1, semi1)

                @pl.when(k + 2 < cnt)
                def _():
                    fire(k + 2, stage0, semi0)

                transpose_block(stage1, outb1)
                col1 = (start + k + 1) * TCOL
                pltpu.sync_copy(outb1, out_hbm.at[pl.ds(col1 * D, TCOL * D)])
            return 0

        lax.fori_loop(0, (per + 2) // 2, body2, 0)

        if rem:
            @pl.when(wid == NW - 1)
            def _():
                n = rem * D
                pltpu.sync_copy(tail_hbm, outb0.at[pl.ds(0, n)])
                pltpu.sync_copy(outb0.at[pl.ds(0, n)],
                                out_hbm.at[pl.ds(n_full * TCOL * D, n)])

    return transpose_k


def _make_pool_kernel(B, L, V, D):
    assert B % NW == 0
    b_per_w = B // NW
    chunks = []
    off = 0
    while off < L:
        n = min(128, L - off)
        chunks.append((off, n))
        off += n
    n_j = D // LANES
    inv_l = 1.0 / float(L)

    mesh = plsc.VectorSubcoreMesh(
        core_axis_name="c", subcore_axis_name="s", num_cores=NC,
        num_subcores=NS)

    @functools.partial(
        pl.kernel,
        mesh=mesh,
        compiler_params=pltpu.CompilerParams(use_tc_tiling_on_sc=False),
        out_type=jax.ShapeDtypeStruct((B, D), jnp.float32),
        scratch_types=[
            pltpu.VMEM((b_per_w, L), jnp.int32),
            pltpu.VMEM((L, D), jnp.float32),
            pltpu.VMEM((L, D), jnp.float32),
            pltpu.VMEM((b_per_w, D), jnp.float32),
            pltpu.SemaphoreType.DMA,
            pltpu.SemaphoreType.DMA,
        ],
    )
    def pool(x_hbm, table_hbm, out_hbm, idx_v, buf0, buf1, pooled_v,
             sem0, sem1):
        wid = lax.axis_index("s") * NC + lax.axis_index("c")
        base = wid * b_per_w

        pltpu.sync_copy(x_hbm.at[pl.ds(base, b_per_w)], idx_v)

        def fire(r, buf, sem):
            for (o, n) in chunks:
                pltpu.async_copy(
                    table_hbm.at[idx_v.at[r, pl.ds(o, n)]],
                    buf.at[pl.ds(o, n)], sem)

        def drain(r, buf, sem):
            for (o, n) in chunks:
                pltpu.make_async_copy(
                    table_hbm.at[idx_v.at[r, pl.ds(o, n)]],
                    buf.at[pl.ds(o, n)], sem).wait()

        def accum(r, buf):
            def body(s, accs):
                return tuple(
                    a + buf[s, pl.ds(j * LANES, LANES)]
                    for j, a in enumerate(accs))
            accs = lax.fori_loop(
                0, L, body,
                tuple(jnp.zeros((LANES,), jnp.float32) for _ in range(n_j)))
            for j in range(n_j):
                pooled_v[r, pl.ds(j * LANES, LANES)] = accs[j] * inv_l

        fire(0, buf0, sem0)

        def body2(i, _):
            r = i * 2
            drain(r, buf0, sem0)
            fire(r + 1, buf1, sem1)
            accum(r, buf0)
            drain(r + 1, buf1, sem1)

            @pl.when(r + 2 < b_per_w)
            def _():
                fire(r + 2, buf0, sem0)

            accum(r + 1, buf1)
            return 0

        lax.fori_loop(0, b_per_w // 2, body2, 0)

        pltpu.sync_copy(pooled_v, out_hbm.at[pl.ds(base, b_per_w)])

    return pool


def _mm_body(p_ref, w_ref, b_ref, o_ref):
    o_ref[...] = lax.dot_general(
        p_ref[...], w_ref[...],
        dimension_numbers=(((1,), (1,)), ((), ())),
        preferred_element_type=jnp.float32) + b_ref[...]


def kernel(x, table, W, b):
    B, L = x.shape
    V, D = table.shape
    C = W.shape[0]

    rem = V % 128
    tail = table[V - rem:, :].reshape(rem * D)
    table_lin = _make_transpose_kernel(V, D)(table.T, tail)
    pooled = _make_pool_kernel(B, L, V, D)(
        x.astype(jnp.int32), table_lin.reshape(V, D))

    logit = pl.pallas_call(
        _mm_body,
        out_shape=jax.ShapeDtypeStruct((B, C), jnp.float32),
    )(pooled, W, b.reshape(1, C))
    return logit


# speedup vs baseline: 1.3163x; 1.3163x over previous
import functools

import jax
import jax.numpy as jnp
from jax import lax
from jax.experimental import pallas as pl
from jax.experimental.pallas import tpu as pltpu
from jax.experimental.pallas import tpu_sc as plsc

NC = 2
NS = 16
NW = NC * NS
LANES = 16


def _make_transpose_kernel(V, D):
    TCOL = 128
    n_full = V // TCOL
    rem = V - n_full * TCOL
    per = n_full // NW
    extra = n_full - per * NW
    n_j = D // LANES

    mesh = plsc.VectorSubcoreMesh(
        core_axis_name="c", subcore_axis_name="s", num_cores=NC,
        num_subcores=NS)

    @functools.partial(
        pl.kernel,
        mesh=mesh,
        compiler_params=pltpu.CompilerParams(
            use_tc_tiling_on_sc=True, needs_layout_passes=False),
        out_type=jax.ShapeDtypeStruct((V * D,), jnp.float32),
        scratch_types=[
            pltpu.VMEM((D, TCOL), jnp.float32),
            pltpu.VMEM((D, TCOL), jnp.float32),
            pltpu.VMEM((TCOL * D,), jnp.float32),
            pltpu.VMEM((TCOL * D,), jnp.float32),
            pltpu.SemaphoreType.DMA,
            pltpu.SemaphoreType.DMA,
            pltpu.SemaphoreType.DMA,
        ],
    )
    def transpose_k(tT_hbm, tail_hbm, out_hbm, stage0, stage1, outb0, outb1,
                    semi0, semi1, semo):
        wid = lax.axis_index("s") * NC + lax.axis_index("c")
        cnt = per + jnp.where(wid < extra, 1, 0)
        start = per * wid + jnp.minimum(wid, extra)

        iota = lax.iota(jnp.int32, 16)
        idx_base = [(iota + LANES * g) * D for g in range(TCOL // LANES)]
        zvec = jnp.zeros((LANES,), jnp.int32) + wid * 0

        def fire(k, stage, sem):
            col0 = (start + k) * TCOL
            pltpu.async_copy(
                tT_hbm.at[:, pl.ds(col0, TCOL)], stage, sem)

        def drain(k, stage, sem):
            col0 = (start + k) * TCOL
            pltpu.make_async_copy(
                tT_hbm.at[:, pl.ds(col0, TCOL)], stage, sem).wait()

        def transpose_block(stage, outb):
            @plsc.parallel_loop(0, D, unroll=8)
            def dbody(d):
                for g in range(TCOL // LANES):
                    v = stage[d, pl.ds(LANES * g, LANES)]
                    plsc.store_scatter(outb, [idx_base[g] + d], v)

        fire(0, stage0, semi0)

        def body2(i, _):
            k = i * 2

            @pl.when(k < cnt)
            def _():
                drain(k, stage0, semi0)

                @pl.when(k + 1 < cnt)
                def _():
                    fire(k + 1, stage1, semi1)

                transpose_block(stage0, outb0)
                col0 = (start + k) * TCOL
                pltpu.sync_copy(outb0, out_hbm.at[pl.ds(col0 * D, TCOL * D)])

            @pl.when(k + 1 < cnt)
            def _():
                drain(k + 1, stage1, semi1)

                @pl.when(k + 2 < cnt)
                def _():
                    fire(k + 2, stage0, semi0)

                transpose_block(stage1, outb1)
                col1 = (start + k + 1) * TCOL
                pltpu.sync_copy(outb1, out_hbm.at[pl.ds(col1 * D, TCOL * D)])
            return 0

        lax.fori_loop(0, (per + 2) // 2, body2, 0)

        if rem:
            @pl.when(wid == NW - 1)
            def _():
                n = rem * D
                pltpu.sync_copy(tail_hbm, outb0.at[pl.ds(0, n)])
                pltpu.sync_copy(outb0.at[pl.ds(0, n)],
                                out_hbm.at[pl.ds(n_full * TCOL * D, n)])

    return transpose_k


def _make_pool_kernel(B, L, V, D):
    assert B % NW == 0
    b_per_w = B // NW
    chunks = []
    off = 0
    while off < L:
        n = min(128, L - off)
        chunks.append((off, n))
        off += n
    n_j = D // LANES
    inv_l = 1.0 / float(L)

    mesh = plsc.VectorSubcoreMesh(
        core_axis_name="c", subcore_axis_name="s", num_cores=NC,
        num_subcores=NS)

    @functools.partial(
        pl.kernel,
        mesh=mesh,
        compiler_params=pltpu.CompilerParams(use_tc_tiling_on_sc=False),
        out_type=jax.ShapeDtypeStruct((B, D), jnp.float32),
        scratch_types=[
            pltpu.VMEM((b_per_w, L), jnp.int32),
            pltpu.VMEM((L, D), jnp.float32),
            pltpu.VMEM((L, D), jnp.float32),
            pltpu.VMEM((b_per_w, D), jnp.float32),
            pltpu.SemaphoreType.DMA,
            pltpu.SemaphoreType.DMA,
        ],
    )
    def pool(x_hbm, table_hbm, out_hbm, idx_v, buf0, buf1, pooled_v,
             sem0, sem1):
        wid = lax.axis_index("s") * NC + lax.axis_index("c")
        base = wid * b_per_w

        pltpu.sync_copy(x_hbm.at[pl.ds(base, b_per_w)], idx_v)

        def fire(r, buf, sem):
            for (o, n) in chunks:
                pltpu.async_copy(
                    table_hbm.at[idx_v.at[r, pl.ds(o, n)]],
                    buf.at[pl.ds(o, n)], sem)

        def drain(r, buf, sem):
            for (o, n) in chunks:
                pltpu.make_async_copy(
                    table_hbm.at[idx_v.at[r, pl.ds(o, n)]],
                    buf.at[pl.ds(o, n)], sem).wait()

        def accum(r, buf):
            def body(s, accs):
                return tuple(
                    a + buf[s, pl.ds(j * LANES, LANES)]
                    for j, a in enumerate(accs))
            accs = lax.fori_loop(
                0, L, body,
                tuple(jnp.zeros((LANES,), jnp.float32) for _ in range(n_j)))
            for j in range(n_j):
                pooled_v[r, pl.ds(j * LANES, LANES)] = accs[j] * inv_l

        fire(0, buf0, sem0)

        def body2(i, _):
            r = i * 2
            drain(r, buf0, sem0)
            fire(r + 1, buf1, sem1)
            accum(r, buf0)
            drain(r + 1, buf1, sem1)

            @pl.when(r + 2 < b_per_w)
            def _():
                fire(r + 2, buf0, sem0)

            accum(r + 1, buf1)
            return 0

        lax.fori_loop(0, b_per_w // 2, body2, 0)

        pltpu.sync_copy(pooled_v, out_hbm.at[pl.ds(base, b_per_w)])

    return pool


def _mm_body(p_ref, w_ref, b_ref, o_ref):
    o_ref[...] = lax.dot_general(
        p_ref[...], w_ref[...],
        dimension_numbers=(((1,), (1,)), ((), ())),
        preferred_element_type=jnp.float32) + b_ref[...]


def kernel(x, table, W, b):
    B, L = x.shape
    V, D = table.shape
    C = W.shape[0]

    rem = V % 128
    tail = table[V - rem:, :].reshape(rem * D)
    table_lin = _make_transpose_kernel(V, D)(table.T, tail)
    pooled = _make_pool_kernel(B, L, V, D)(
        x.astype(jnp.int32), table_lin.reshape(V, D))

    logit = pl.pallas_call(
        _mm_body,
        out_shape=jax.ShapeDtypeStruct((B, C), jnp.float32),
    )(pooled, W, b.reshape(1, C))
    return logit


# speedup vs baseline: 3.1036x; 2.3578x over previous
import functools

import jax
import jax.numpy as jnp
from jax import lax
from jax.experimental import pallas as pl
from jax.experimental.pallas import tpu as pltpu
from jax.experimental.pallas import tpu_sc as plsc

NC = 2
NS = 16
NW = NC * NS
LANES = 16


def _make_transpose_kernel(V, D):
    TCOL = 128
    n_full = V // TCOL
    rem = V - n_full * TCOL
    per = n_full // NW
    extra = n_full - per * NW
    n_j = D // LANES

    mesh = plsc.VectorSubcoreMesh(
        core_axis_name="c", subcore_axis_name="s", num_cores=NC,
        num_subcores=NS)

    @functools.partial(
        pl.kernel,
        mesh=mesh,
        compiler_params=pltpu.CompilerParams(
            use_tc_tiling_on_sc=True, needs_layout_passes=False),
        out_type=jax.ShapeDtypeStruct((V * D,), jnp.float32),
        scratch_types=[
            pltpu.VMEM((D, TCOL), jnp.float32),
            pltpu.VMEM((D, TCOL), jnp.float32),
            pltpu.VMEM((D * 136,), jnp.float32),
            pltpu.VMEM((TCOL * D,), jnp.float32),
            pltpu.VMEM((TCOL * D,), jnp.float32),
            pltpu.SemaphoreType.DMA,
            pltpu.SemaphoreType.DMA,
            pltpu.SemaphoreType.DMA,
        ],
    )
    def transpose_k(tT_hbm, tail_hbm, out_hbm, stage0, stage1, skew, outb0,
                    outb1, semi0, semi1, semo):
        wid = lax.axis_index("s") * NC + lax.axis_index("c")
        cnt = per + jnp.where(wid < extra, 1, 0)
        start = per * wid + jnp.minimum(wid, extra)

        iota = lax.iota(jnp.int32, 16)
        skew_base = [(iota + LANES * j) * 136 for j in range(n_j)]

        def fire(k, stage, sem):
            col0 = (start + k) * TCOL
            pltpu.async_copy(
                tT_hbm.at[:, pl.ds(col0, TCOL)], stage, sem)

        def drain(k, stage, sem):
            col0 = (start + k) * TCOL
            pltpu.make_async_copy(
                tT_hbm.at[:, pl.ds(col0, TCOL)], stage, sem).wait()

        def transpose_block(stage, outb):
            @plsc.parallel_loop(0, D, unroll=4)
            def dbody(d):
                for g in range(TCOL // LANES):
                    skew[pl.ds(d * 136 + LANES * g, LANES)] = (
                        stage[d, pl.ds(LANES * g, LANES)])

            @plsc.parallel_loop(0, TCOL, unroll=8)
            def cbody(c):
                base = c * D
                for j in range(n_j):
                    v = plsc.load_gather(skew, [skew_base[j] + c])
                    outb[pl.ds(base + LANES * j, LANES)] = v

        fire(0, stage0, semi0)

        def body2(i, _):
            k = i * 2

            @pl.when(k < cnt)
            def _():
                drain(k, stage0, semi0)

                @pl.when(k + 1 < cnt)
                def _():
                    fire(k + 1, stage1, semi1)

                transpose_block(stage0, outb0)
                col0 = (start + k) * TCOL
                pltpu.sync_copy(outb0, out_hbm.at[pl.ds(col0 * D, TCOL * D)])

            @pl.when(k + 1 < cnt)
            def _():
                drain(k + 1, stage1, semi1)

                @pl.when(k + 2 < cnt)
                def _():
                    fire(k + 2, stage0, semi0)

                transpose_block(stage1, outb1)
                col1 = (start + k + 1) * TCOL
                pltpu.sync_copy(outb1, out_hbm.at[pl.ds(col1 * D, TCOL * D)])
            return 0

        lax.fori_loop(0, (per + 2) // 2, body2, 0)

        if rem:
            @pl.when(wid == NW - 1)
            def _():
                n = rem * D
                pltpu.sync_copy(tail_hbm, outb0.at[pl.ds(0, n)])
                pltpu.sync_copy(outb0.at[pl.ds(0, n)],
                                out_hbm.at[pl.ds(n_full * TCOL * D, n)])

    return transpose_k


def _make_pool_kernel(B, L, V, D):
    assert B % NW == 0
    b_per_w = B // NW
    chunks = []
    off = 0
    while off < L:
        n = min(128, L - off)
        chunks.append((off, n))
        off += n
    n_j = D // LANES
    inv_l = 1.0 / float(L)

    mesh = plsc.VectorSubcoreMesh(
        core_axis_name="c", subcore_axis_name="s", num_cores=NC,
        num_subcores=NS)

    @functools.partial(
        pl.kernel,
        mesh=mesh,
        compiler_params=pltpu.CompilerParams(use_tc_tiling_on_sc=False),
        out_type=jax.ShapeDtypeStruct((B, D), jnp.float32),
        scratch_types=[
            pltpu.VMEM((b_per_w, L), jnp.int32),
            pltpu.VMEM((L, D), jnp.float32),
            pltpu.VMEM((L, D), jnp.float32),
            pltpu.VMEM((b_per_w, D), jnp.float32),
            pltpu.SemaphoreType.DMA,
            pltpu.SemaphoreType.DMA,
        ],
    )
    def pool(x_hbm, table_hbm, out_hbm, idx_v, buf0, buf1, pooled_v,
             sem0, sem1):
        wid = lax.axis_index("s") * NC + lax.axis_index("c")
        base = wid * b_per_w

        pltpu.sync_copy(x_hbm.at[pl.ds(base, b_per_w)], idx_v)

        def fire(r, buf, sem):
            for (o, n) in chunks:
                pltpu.async_copy(
                    table_hbm.at[idx_v.at[r, pl.ds(o, n)]],
                    buf.at[pl.ds(o, n)], sem)

        def drain(r, buf, sem):
            for (o, n) in chunks:
                pltpu.make_async_copy(
                    table_hbm.at[idx_v.at[r, pl.ds(o, n)]],
                    buf.at[pl.ds(o, n)], sem).wait()

        def accum(r, buf):
            def body(s, accs):
                return tuple(
                    a + buf[s, pl.ds(j * LANES, LANES)]
                    for j, a in enumerate(accs))
            accs = lax.fori_loop(
                0, L, body,
                tuple(jnp.zeros((LANES,), jnp.float32) for _ in range(n_j)))
            for j in range(n_j):
                pooled_v[r, pl.ds(j * LANES, LANES)] = accs[j] * inv_l

        fire(0, buf0, sem0)

        def body2(i, _):
            r = i * 2
            drain(r, buf0, sem0)
            fire(r + 1, buf1, sem1)
            accum(r, buf0)
            drain(r + 1, buf1, sem1)

            @pl.when(r + 2 < b_per_w)
            def _():
                fire(r + 2, buf0, sem0)

            accum(r + 1, buf1)
            return 0

        lax.fori_loop(0, b_per_w // 2, body2, 0)

        pltpu.sync_copy(pooled_v, out_hbm.at[pl.ds(base, b_per_w)])

    return pool


def _mm_body(p_ref, w_ref, b_ref, o_ref):
    o_ref[...] = lax.dot_general(
        p_ref[...], w_ref[...],
        dimension_numbers=(((1,), (1,)), ((), ())),
        preferred_element_type=jnp.float32) + b_ref[...]


def kernel(x, table, W, b):
    B, L = x.shape
    V, D = table.shape
    C = W.shape[0]

    rem = V % 128
    tail = table[V - rem:, :].reshape(rem * D)
    table_lin = _make_transpose_kernel(V, D)(table.T, tail)
    pooled = _make_pool_kernel(B, L, V, D)(
        x.astype(jnp.int32), table_lin.reshape(V, D))

    logit = pl.pallas_call(
        _mm_body,
        out_shape=jax.ShapeDtypeStruct((B, C), jnp.float32),
    )(pooled, W, b.reshape(1, C))
    return logit


# speedup vs baseline: 3.1062x; 1.0008x over previous
import functools

import jax
import jax.numpy as jnp
from jax import lax
from jax.experimental import pallas as pl
from jax.experimental.pallas import tpu as pltpu
from jax.experimental.pallas import tpu_sc as plsc

NC = 2
NS = 16
NW = NC * NS
LANES = 16


def _make_transpose_kernel(V, D):
    TCOL = 128
    n_full = V // TCOL
    rem = V - n_full * TCOL
    per = n_full // NW
    extra = n_full - per * NW
    n_j = D // LANES

    mesh = plsc.VectorSubcoreMesh(
        core_axis_name="c", subcore_axis_name="s", num_cores=NC,
        num_subcores=NS)

    @functools.partial(
        pl.kernel,
        mesh=mesh,
        compiler_params=pltpu.CompilerParams(
            use_tc_tiling_on_sc=True, needs_layout_passes=False),
        out_type=jax.ShapeDtypeStruct((V * D,), jnp.float32),
        scratch_types=[
            pltpu.VMEM((D, TCOL), jnp.float32),
            pltpu.VMEM((D, TCOL), jnp.float32),
            pltpu.VMEM((D * 136,), jnp.float32),
            pltpu.VMEM((TCOL * D,), jnp.float32),
            pltpu.VMEM((TCOL * D,), jnp.float32),
            pltpu.SemaphoreType.DMA,
            pltpu.SemaphoreType.DMA,
            pltpu.SemaphoreType.DMA,
        ],
    )
    def transpose_k(tT_hbm, tail_hbm, out_hbm, stage0, stage1, skew, outb0,
                    outb1, semi0, semi1, semo):
        wid = lax.axis_index("s") * NC + lax.axis_index("c")
        cnt = per + jnp.where(wid < extra, 1, 0)
        start = per * wid + jnp.minimum(wid, extra)

        iota = lax.iota(jnp.int32, 16)
        skew_base = [(iota + LANES * j) * 136 for j in range(n_j)]

        def fire(k, stage, sem):
            col0 = (start + k) * TCOL
            pltpu.async_copy(
                tT_hbm.at[:, pl.ds(col0, TCOL)], stage, sem)

        def drain(k, stage, sem):
            col0 = (start + k) * TCOL
            pltpu.make_async_copy(
                tT_hbm.at[:, pl.ds(col0, TCOL)], stage, sem).wait()

        def transpose_block(stage, outb):
            @plsc.parallel_loop(0, D, unroll=4)
            def dbody(d):
                for g in range(TCOL // LANES):
                    skew[pl.ds(d * 136 + LANES * g, LANES)] = (
                        stage[d, pl.ds(LANES * g, LANES)])

            @plsc.parallel_loop(0, TCOL, unroll=8)
            def cbody(c):
                base = c * D
                for j in range(n_j):
                    v = plsc.load_gather(skew, [skew_base[j] + c])
                    outb[pl.ds(base + LANES * j, LANES)] = v

        fire(0, stage0, semi0)

        def body2(i, _):
            k = i * 2

            @pl.when(k < cnt)
            def _():
                drain(k, stage0, semi0)

                @pl.when(k + 1 < cnt)
                def _():
                    fire(k + 1, stage1, semi1)

                transpose_block(stage0, outb0)
                col0 = (start + k) * TCOL
                pltpu.sync_copy(outb0, out_hbm.at[pl.ds(col0 * D, TCOL * D)])

            @pl.when(k + 1 < cnt)
            def _():
                drain(k + 1, stage1, semi1)

                @pl.when(k + 2 < cnt)
                def _():
                    fire(k + 2, stage0, semi0)

                transpose_block(stage1, outb1)
                col1 = (start + k + 1) * TCOL
                pltpu.sync_copy(outb1, out_hbm.at[pl.ds(col1 * D, TCOL * D)])
            return 0

        lax.fori_loop(0, (per + 2) // 2, body2, 0)

        if rem:
            @pl.when(wid == NW - 1)
            def _():
                n = rem * D
                pltpu.sync_copy(tail_hbm, outb0.at[pl.ds(0, n)])
                pltpu.sync_copy(outb0.at[pl.ds(0, n)],
                                out_hbm.at[pl.ds(n_full * TCOL * D, n)])

    return transpose_k


def _make_pool_kernel(B, L, V, D):
    assert B % NW == 0
    b_per_w = B // NW
    chunks = []
    off = 0
    while off < L:
        n = min(128, L - off)
        chunks.append((off, n))
        off += n
    n_j = D // LANES
    inv_l = 1.0 / float(L)

    mesh = plsc.VectorSubcoreMesh(
        core_axis_name="c", subcore_axis_name="s", num_cores=NC,
        num_subcores=NS)

    @functools.partial(
        pl.kernel,
        mesh=mesh,
        compiler_params=pltpu.CompilerParams(use_tc_tiling_on_sc=False),
        out_type=jax.ShapeDtypeStruct((B, D), jnp.float32),
        scratch_types=[
            pltpu.VMEM((b_per_w, L), jnp.int32),
            pltpu.VMEM((L, D), jnp.float32),
            pltpu.VMEM((L, D), jnp.float32),
            pltpu.VMEM((b_per_w, D), jnp.float32),
            pltpu.SemaphoreType.DMA,
            pltpu.SemaphoreType.DMA,
        ],
    )
    def pool(x_hbm, table_hbm, out_hbm, idx_v, buf0, buf1, pooled_v,
             sem0, sem1):
        wid = lax.axis_index("s") * NC + lax.axis_index("c")
        base = wid * b_per_w

        pltpu.sync_copy(x_hbm.at[pl.ds(base, b_per_w)], idx_v)

        def fire(r, buf, sem):
            for (o, n) in chunks:
                pltpu.async_copy(
                    table_hbm.at[idx_v.at[r, pl.ds(o, n)]],
                    buf.at[pl.ds(o, n)], sem)

        def drain(r, buf, sem):
            for (o, n) in chunks:
                pltpu.make_async_copy(
                    table_hbm.at[idx_v.at[r, pl.ds(o, n)]],
                    buf.at[pl.ds(o, n)], sem).wait()

        def accum(r, buf):
            zero = tuple(jnp.zeros((LANES,), jnp.float32) for _ in range(n_j))

            def body(s, accs):
                return tuple(
                    a + buf[s, pl.ds(j * LANES, LANES)]
                    for j, a in enumerate(accs))
            accs = plsc.parallel_loop(0, L, unroll=8, carry=zero)(body)
            for j in range(n_j):
                pooled_v[r, pl.ds(j * LANES, LANES)] = accs[j] * inv_l

        fire(0, buf0, sem0)

        def body2(i, _):
            r = i * 2
            drain(r, buf0, sem0)
            fire(r + 1, buf1, sem1)
            accum(r, buf0)
            drain(r + 1, buf1, sem1)

            @pl.when(r + 2 < b_per_w)
            def _():
                fire(r + 2, buf0, sem0)

            accum(r + 1, buf1)
            return 0

        lax.fori_loop(0, b_per_w // 2, body2, 0)

        pltpu.sync_copy(pooled_v, out_hbm.at[pl.ds(base, b_per_w)])

    return pool


def _mm_body(p_ref, w_ref, b_ref, o_ref):
    o_ref[...] = lax.dot_general(
        p_ref[...], w_ref[...],
        dimension_numbers=(((1,), (1,)), ((), ())),
        preferred_element_type=jnp.float32) + b_ref[...]


def kernel(x, table, W, b):
    B, L = x.shape
    V, D = table.shape
    C = W.shape[0]

    rem = V % 128
    tail = table[V - rem:, :].reshape(rem * D)
    table_lin = _make_transpose_kernel(V, D)(table.T, tail)
    pooled = _make_pool_kernel(B, L, V, D)(
        x.astype(jnp.int32), table_lin.reshape(V, D))

    logit = pl.pallas_call(
        _mm_body,
        out_shape=jax.ShapeDtypeStruct((B, C), jnp.float32),
    )(pooled, W, b.reshape(1, C))
    return logit
